# baseline (speedup 1.0000x reference)
import functools

import jax
import jax.numpy as jnp
from jax import lax
from jax.experimental import pallas as pl
from jax.experimental.pallas import tpu as pltpu
from jax.experimental.pallas import tpu_sc as plsc

N_USERS = 1000000
N_ITEMS = 1000000
K = 32
BATCH = 16384

NC = 2
NS = 16
NW = NC * NS
B_PER_W = BATCH // NW
CHUNK = 128
N_CHUNKS = B_PER_W // CHUNK
WAVE = 8

_mesh = plsc.VectorSubcoreMesh(core_axis_name="c", subcore_axis_name="s")


@functools.partial(
    pl.kernel,
    out_type=jax.ShapeDtypeStruct((BATCH,), jnp.float32),
    mesh=_mesh,
    compiler_params=pltpu.CompilerParams(use_tc_tiling_on_sc=False),
    scratch_types=[
        pltpu.VMEM((N_CHUNKS, CHUNK), jnp.int32),
        pltpu.VMEM((N_CHUNKS, CHUNK), jnp.int32),
        pltpu.VMEM((K, B_PER_W), jnp.float32),
        pltpu.VMEM((K, B_PER_W), jnp.float32),
        pltpu.VMEM((B_PER_W,), jnp.float32),
        pltpu.SemaphoreType.DMA,
    ],
)
def _pmf_kernel(uidx_hbm, iidx_hbm, utab_hbm, itab_hbm, out_hbm,
                uidx_v, iidx_v, urows_v, irows_v, out_v, sem):
    wid = lax.axis_index("s") * NC + lax.axis_index("c")
    base = wid * B_PER_W

    for c in range(N_CHUNKS):
        pltpu.sync_copy(uidx_hbm.at[pl.ds(base + c * CHUNK, CHUNK)],
                        uidx_v.at[c])
        pltpu.sync_copy(iidx_hbm.at[pl.ds(base + c * CHUNK, CHUNK)],
                        iidx_v.at[c])

    for k0 in range(0, K, WAVE):
        copies = []
        for k in range(k0, k0 + WAVE):
            for c in range(N_CHUNKS):
                copies.append(pltpu.async_copy(
                    utab_hbm.at[k].at[uidx_v.at[c]],
                    urows_v.at[k, pl.ds(c * CHUNK, CHUNK)], sem))
                copies.append(pltpu.async_copy(
                    itab_hbm.at[k].at[iidx_v.at[c]],
                    irows_v.at[k, pl.ds(c * CHUNK, CHUNK)], sem))
        for cp in copies:
            cp.wait()

    def grp_body(g, _):
        s = pl.ds(g * 16, 16)
        acc = urows_v[0, s] * irows_v[0, s]
        for k in range(1, K):
            acc = acc + urows_v[k, s] * irows_v[k, s]
        out_v[s] = acc
        return 0

    lax.fori_loop(0, B_PER_W // 16, grp_body, 0)

    pltpu.sync_copy(out_v, out_hbm.at[pl.ds(base, B_PER_W)])


def kernel(users_index, items_index, user_embeddings, item_embeddings):
    return _pmf_kernel(users_index.astype(jnp.int32),
                       items_index.astype(jnp.int32),
                       user_embeddings.T, item_embeddings.T)


# speedup vs baseline: 2.5904x; 2.5904x over previous
import functools

import jax
import jax.numpy as jnp
from jax import lax
from jax.experimental import pallas as pl
from jax.experimental.pallas import tpu as pltpu
from jax.experimental.pallas import tpu_sc as plsc

N_USERS = 1000000
N_ITEMS = 1000000
K = 32
BATCH = 16384

NC = 2
NS = 16
NW = NC * NS
B_PER_W = BATCH // NW
CHUNK = 128
N_CHUNKS = B_PER_W // CHUNK
RPT = 8
LOG2_RPT = 3
KW = K // 2
ROW_W = KW * RPT

_mesh = plsc.VectorSubcoreMesh(core_axis_name="c", subcore_axis_name="s")

_GATHER_DNUMS = lax.GatherDimensionNumbers(
    offset_dims=(), collapsed_slice_dims=(0,), start_index_map=(0,))


def _vperm(x, idx):
    return lax.gather(x, idx[:, None], _GATHER_DNUMS, slice_sizes=(1,),
                      mode=lax.GatherScatterMode.PROMISE_IN_BOUNDS)


@functools.partial(
    pl.kernel,
    out_type=jax.ShapeDtypeStruct((BATCH,), jnp.float32),
    mesh=_mesh,
    compiler_params=pltpu.CompilerParams(needs_layout_passes=False),
    scratch_types=[
        pltpu.VMEM((N_CHUNKS, CHUNK), jnp.int32),
        pltpu.VMEM((N_CHUNKS, CHUNK), jnp.int32),
        pltpu.VMEM((N_CHUNKS, CHUNK), jnp.int32),
        pltpu.VMEM((N_CHUNKS, CHUNK), jnp.int32),
        pltpu.VMEM((CHUNK, ROW_W), jnp.int32),
        pltpu.VMEM((CHUNK, ROW_W), jnp.int32),
        pltpu.VMEM((B_PER_W,), jnp.float32),
        pltpu.SemaphoreType.DMA,
    ],
)
def _pmf_kernel(uidx_hbm, iidx_hbm, utab_hbm, itab_hbm, out_hbm,
                uidx_v, iidx_v, utix_v, itix_v, urows_v, irows_v,
                out_v, sem):
    wid = lax.axis_index("s") * NC + lax.axis_index("c")
    base = wid * B_PER_W

    for c in range(N_CHUNKS):
        pltpu.sync_copy(uidx_hbm.at[pl.ds(base + c * CHUNK, CHUNK)],
                        uidx_v.at[c])
        pltpu.sync_copy(iidx_hbm.at[pl.ds(base + c * CHUNK, CHUNK)],
                        iidx_v.at[c])
    for c in range(N_CHUNKS):
        for j in range(CHUNK // 16):
            s = pl.ds(j * 16, 16)
            utix_v[c, s] = uidx_v[c, s] >> LOG2_RPT
            itix_v[c, s] = iidx_v[c, s] >> LOG2_RPT

    lane = lax.iota(jnp.int32, 16)
    perms = [lane ^ (1 << sft) for sft in range(4)]

    def lane_sum(x):
        for p in perms:
            x = x + _vperm(x, p)
        return x

    for c in range(N_CHUNKS):
        ucp = pltpu.async_copy(utab_hbm.at[utix_v.at[c]], urows_v, sem)
        icp = pltpu.async_copy(itab_hbm.at[itix_v.at[c]], irows_v, sem)
        ucp.wait()
        icp.wait()

        def grp_body(g, _):
            rr0 = g * 16
            uqv = (uidx_v[c, pl.ds(rr0, 16)] & (RPT - 1)) * KW
            iqv = (iidx_v[c, pl.ds(rr0, 16)] & (RPT - 1)) * KW
            acc = jnp.zeros((16,), jnp.float32)
            for r in range(16):
                i = rr0 + r
                uq = pl.multiple_of(uqv[r], KW)
                iq = pl.multiple_of(iqv[r], KW)
                ubf = plsc.bitcast(urows_v[i, pl.ds(uq, KW)], jnp.bfloat16)
                vbf = plsc.bitcast(irows_v[i, pl.ds(iq, KW)], jnp.bfloat16)
                ua, ub = plsc.unpack(ubf, format=plsc.PackFormat.INTERLEAVED)
                va, vb = plsc.unpack(vbf, format=plsc.PackFormat.INTERLEAVED)
                acc = jnp.where(lane == r, lane_sum(ua * va + ub * vb), acc)
            out_v[pl.ds(c * CHUNK + rr0, 16)] = acc
            return 0

        lax.fori_loop(0, CHUNK // 16, grp_body, 0)

    pltpu.sync_copy(out_v, out_hbm.at[pl.ds(base, B_PER_W)])


def kernel(users_index, items_index, user_embeddings, item_embeddings):
    def pack(t, n):
        tb = t.astype(jnp.bfloat16).reshape(n, KW, 2)
        return lax.bitcast_convert_type(tb, jnp.int32).reshape(
            n // RPT, ROW_W)

    return _pmf_kernel(users_index.astype(jnp.int32),
                       items_index.astype(jnp.int32),
                       pack(user_embeddings, N_USERS),
                       pack(item_embeddings, N_ITEMS))


# speedup vs baseline: 5.7118x; 2.2050x over previous
import functools

import jax
import jax.numpy as jnp
from jax import lax
from jax.experimental import pallas as pl
from jax.experimental.pallas import tpu as pltpu
from jax.experimental.pallas import tpu_sc as plsc

N_USERS = 1000000
N_ITEMS = 1000000
K = 32
BATCH = 16384

NC = 2
NS = 16
NW = NC * NS
B_PER_W = BATCH // NW
CHUNK = 128
N_CHUNKS = B_PER_W // CHUNK

_mesh = plsc.VectorSubcoreMesh(core_axis_name="c", subcore_axis_name="s")

_GATHER_DNUMS = lax.GatherDimensionNumbers(
    offset_dims=(), collapsed_slice_dims=(0,), start_index_map=(0,))


def _vperm(x, idx):
    return lax.gather(x, idx[:, None], _GATHER_DNUMS, slice_sizes=(1,),
                      mode=lax.GatherScatterMode.PROMISE_IN_BOUNDS)


@functools.partial(
    pl.kernel,
    out_type=jax.ShapeDtypeStruct((BATCH,), jnp.float32),
    mesh=_mesh,
    compiler_params=pltpu.CompilerParams(use_tc_tiling_on_sc=False),
    scratch_types=[
        pltpu.VMEM((N_CHUNKS, CHUNK), jnp.int32),
        pltpu.VMEM((N_CHUNKS, CHUNK), jnp.int32),
        pltpu.VMEM((B_PER_W, K), jnp.float32),
        pltpu.VMEM((B_PER_W, K), jnp.float32),
        pltpu.VMEM((B_PER_W,), jnp.float32),
        pltpu.SemaphoreType.DMA,
    ],
)
def _pmf_kernel(uidx_hbm, iidx_hbm, utab_hbm, itab_hbm, out_hbm,
                uidx_v, iidx_v, urows_v, irows_v, out_v, sem):
    wid = lax.axis_index("s") * NC + lax.axis_index("c")
    base = wid * B_PER_W

    for j in range(N_CHUNKS):
        pltpu.sync_copy(uidx_hbm.at[pl.ds(base + j * CHUNK, CHUNK)],
                        uidx_v.at[j])
        pltpu.sync_copy(iidx_hbm.at[pl.ds(base + j * CHUNK, CHUNK)],
                        iidx_v.at[j])

    copies = []
    for j in range(N_CHUNKS):
        copies.append(pltpu.async_copy(
            utab_hbm.at[uidx_v.at[j]],
            urows_v.at[pl.ds(j * CHUNK, CHUNK)], sem))
        copies.append(pltpu.async_copy(
            itab_hbm.at[iidx_v.at[j]],
            irows_v.at[pl.ds(j * CHUNK, CHUNK)], sem))
    for c in copies:
        c.wait()

    lane = lax.iota(jnp.int32, 16)
    perms = [lane ^ (1 << s) for s in range(4)]

    def lane_sum(x):
        for p in perms:
            x = x + _vperm(x, p)
        return x

    def grp_body(g, _):
        acc = jnp.zeros((16,), jnp.float32)
        for r in range(16):
            i = g * 16 + r
            s = lane_sum(
                urows_v[i, pl.ds(0, 16)] * irows_v[i, pl.ds(0, 16)]
                + urows_v[i, pl.ds(16, 16)] * irows_v[i, pl.ds(16, 16)])
            acc = jnp.where(lane == r, s, acc)
        out_v[pl.ds(g * 16, 16)] = acc
        return 0

    lax.fori_loop(0, B_PER_W // 16, grp_body, 0)

    pltpu.sync_copy(out_v, out_hbm.at[pl.ds(base, B_PER_W)])


def kernel(users_index, items_index, user_embeddings, item_embeddings):
    return _pmf_kernel(users_index.astype(jnp.int32),
                       items_index.astype(jnp.int32),
                       user_embeddings, item_embeddings)
